# Initial kernel scaffold; baseline (speedup 1.0000x reference)
import jax
import jax.numpy as jnp
from jax.experimental import pallas as pl


def kernel(input_seq, word_table, pos_table):
    raise NotImplementedError("write your pallas kernel here")


# speedup vs baseline: 3.4874x; 3.4874x over previous
import jax
import jax.numpy as jnp
from jax import lax
from jax.experimental import pallas as pl
from jax.experimental.pallas import tpu as pltpu
from jax.experimental.pallas import tpu_sc as plsc

B, L, H = 1024, 200, 128
NW = 32
BPW = B // NW
LANES = 16
VPR = H // LANES
CH = (104, 96)
OFF = (0, 104)


def _body(seq_hbm, word_hbm, pos_hbm, out_hbm, idx_v, pos_v, rows_v, sem):
    wid = lax.axis_index("s") * 2 + lax.axis_index("c")
    pltpu.sync_copy(pos_hbm, pos_v)

    def batch_loop(i, carry):
        row0 = (wid * BPW + i) * L
        pltpu.sync_copy(seq_hbm.at[pl.ds(row0, L)], idx_v)
        for j in range(2):
            ch, off = CH[j], OFF[j]
            pltpu.async_copy(
                word_hbm.at[idx_v.at[pl.ds(off, ch)]],
                rows_v.at[pl.ds(0, ch)],
                sem,
            ).wait()

            def row_loop(r, c):
                for k in range(VPR):
                    sl = pl.ds(k * LANES, LANES)
                    rows_v[r, sl] = jnp.maximum(
                        rows_v[r, sl] + pos_v[off + r, sl], 0.0
                    )
                return c

            lax.fori_loop(0, ch, row_loop, 0)
            pltpu.sync_copy(
                rows_v.at[pl.ds(0, ch)],
                out_hbm.at[pl.ds(row0 + off, ch)],
            )
        return carry

    lax.fori_loop(0, BPW, batch_loop, 0)


def kernel(input_seq, word_table, pos_table):
    seq = input_seq.astype(jnp.int32).reshape(B * L)
    mesh = plsc.VectorSubcoreMesh(core_axis_name="c", subcore_axis_name="s")
    f = pl.kernel(
        _body,
        mesh=mesh,
        out_type=jax.ShapeDtypeStruct((B * L, H), jnp.float32),
        scratch_types=[
            pltpu.VMEM((L,), jnp.int32),
            pltpu.VMEM((L, H), jnp.float32),
            pltpu.VMEM((CH[0], H), jnp.float32),
            pltpu.SemaphoreType.DMA,
        ],
    )
    return f(seq, word_table, pos_table).reshape(B, L, H)
